# Initial kernel scaffold; baseline (speedup 1.0000x reference)
import jax
import jax.numpy as jnp
from jax.experimental import pallas as pl


def kernel(pos, x, edge_index, params):
    raise NotImplementedError("write your pallas kernel here")


# speedup vs baseline: 1.0079x; 1.0079x over previous
import functools

import jax
import jax.numpy as jnp
from jax import lax
from jax.experimental import pallas as pl
from jax.experimental.pallas import tpu as pltpu
from jax.experimental.pallas import tpu_sc as plsc

N = 10000
E = 160000
H = 8
EPSV = 1e-5
CDIMS = [(16, 32), (32, 64), (64, 128), (128, 256), (256, 128), (128, 128)]
NSC = 2
NSUB = 16
NTILES = NSC * NSUB
RMAX = 256
EPAD = 164352
EPT = EPAD // NTILES
NP = 10112
ROWS_PER_TILE = NP // NSUB
CSC = 128
WD = CSC
MW = H * CSC

def _dot(a, b):
    return jnp.dot(a, b, preferred_element_type=jnp.float32)


def _softmax8(c8):
    m = jnp.max(c8, axis=1, keepdims=True)
    e = jnp.exp(c8 - m)
    return e / jnp.sum(e, axis=1, keepdims=True)


def _self_msg(M, qc, cout):
    ms = None
    for h in range(H):
        part = M[:, h * CSC:h * CSC + cout]
        ms = qc[0, h] * part if ms is None else ms + qc[0, h] * part
    return ms


def _prestat_body(pos_ref, st_ref):
    p = pos_ref[...]
    mu = jnp.mean(p, axis=0, keepdims=True)
    var = jnp.mean((p - mu) ** 2, axis=0, keepdims=True)
    st_ref[...] = jnp.concatenate([mu, lax.rsqrt(var + EPSV)], axis=0)


def _pre_call(pos, x, g, be, W0, b0, Wc, Up, c8):
    st = pl.pallas_call(
        _prestat_body,
        out_shape=jax.ShapeDtypeStruct((2, 2), jnp.float32),
    )(pos)
    blk = 1000
    grid = N // blk
    cout = CDIMS[0][1]

    def body(pos_ref, x_ref, st_ref, g_ref, be_ref, W0_ref, b0_ref,
             Wc_ref, Up_ref, c8_ref, M_ref, un_ref, ms_ref):
        p = pos_ref[...]
        hn = (p - st_ref[0:1, :]) * st_ref[1:2, :] * g_ref[...] + be_ref[...]
        h0 = jnp.concatenate([hn, x_ref[...]], axis=1)
        h = jnp.maximum(_dot(h0, W0_ref[...]) + b0_ref[...], 0.0)
        M = _dot(h, Wc_ref[...])
        un = jnp.dot(h, Up_ref[...], precision=jax.lax.Precision.HIGHEST,
                     preferred_element_type=jnp.float32)
        qc = _softmax8(c8_ref[...])
        M_ref[...] = M
        for h_ in range(H):
            M_ref[:, (h_ + 1) * CSC - 1:(h_ + 1) * CSC] = jnp.ones(
                (blk, 1), jnp.float32)
        un_ref[...] = un
        ms_ref[...] = _self_msg(M, qc, cout)

    fixed = lambda shape: pl.BlockSpec(shape, lambda i: (0, 0))
    rows = lambda shape: pl.BlockSpec(shape, lambda i: (i, 0))
    return pl.pallas_call(
        body,
        grid=(grid,),
        in_specs=[
            rows((blk, 2)),
            rows((blk, 2)),
            fixed((2, 2)),
            fixed((1, 2)),
            fixed((1, 2)),
            fixed((4, 16)),
            fixed((1, 16)),
            fixed((16, MW)),
            fixed((16, CSC)),
            fixed((1, H)),
        ],
        out_specs=(
            rows((blk, MW)),
            rows((blk, CSC)),
            rows((blk, cout)),
        ),
        out_shape=(
            jax.ShapeDtypeStruct((N, MW), jnp.float32),
            jax.ShapeDtypeStruct((N, CSC), jnp.float32),
            jax.ShapeDtypeStruct((N, cout), jnp.float32),
        ),
    )(pos, x, st, g, be, W0, b0, Wc, Up, c8)


@functools.lru_cache(maxsize=None)
def _q_call(cprev, cout):
    blk = 1000
    grid = N // blk
    nM = 2 if cout == 256 else 1
    mw = nM * MW

    def body(y_ref, st_ref, g_ref, be_ref, Wc_ref, Up_ref, c8_ref, *out_refs):
        y = y_ref[...]
        h = jnp.maximum(
            (y - st_ref[0:1, :]) * st_ref[1:2, :] * g_ref[...] + be_ref[...],
            0.0)
        M = _dot(h, Wc_ref[...])
        un = jnp.dot(h, Up_ref[...], precision=jax.lax.Precision.HIGHEST,
                     preferred_element_type=jnp.float32)
        qc = _softmax8(c8_ref[...])
        if cout == 256:
            Ma_ref, Mb_ref, un_ref, ms_ref = out_refs
            Ma_ref[...] = M[:, :MW]
            Mb_ref[...] = M[:, MW:]
            ms = None
            for h_ in range(H):
                part = jnp.concatenate(
                    [M[:, h_ * CSC:(h_ + 1) * CSC],
                     M[:, MW + h_ * CSC:MW + (h_ + 1) * CSC]], axis=1)
                ms = qc[0, h_] * part if ms is None else ms + qc[0, h_] * part
        else:
            M_ref, un_ref, ms_ref = out_refs
            M_ref[...] = M
            ms = _self_msg(M, qc, cout)
        un_ref[...] = un
        ms_ref[...] = ms

    fixed = lambda shape: pl.BlockSpec(shape, lambda i: (0, 0))
    rows = lambda shape: pl.BlockSpec(shape, lambda i: (i, 0))
    m_specs = [rows((blk, MW)) for _ in range(nM)]
    m_shapes = [jax.ShapeDtypeStruct((N, MW), jnp.float32) for _ in range(nM)]
    return pl.pallas_call(
        body,
        grid=(grid,),
        in_specs=[
            rows((blk, cprev)),
            fixed((2, cprev)),
            fixed((1, cprev)),
            fixed((1, cprev)),
            fixed((cprev, mw)),
            fixed((cprev, CSC)),
            fixed((1, H)),
        ],
        out_specs=tuple(m_specs) + (rows((blk, CSC)), rows((blk, cout))),
        out_shape=tuple(m_shapes) + (
            jax.ShapeDtypeStruct((N, CSC), jnp.float32),
            jax.ShapeDtypeStruct((N, cout), jnp.float32),
        ),
    )


@functools.lru_cache(maxsize=None)
def _p_call(cout, with_cnt):
    two = cout == 256

    def body(*refs):
        if two:
            aggA_ref, aggB_ref, ms_ref, cnt_in_ref, b_ref, y_ref, st_ref = refs
            agg = jnp.concatenate(
                [aggA_ref[0:N, 0:CSC] + aggA_ref[NP:NP + N, 0:CSC],
                 aggB_ref[0:N, 0:CSC] + aggB_ref[NP:NP + N, 0:CSC]], axis=1)
            cnt = cnt_in_ref[...]
        elif with_cnt:
            agg_ref, ms_ref, b_ref, y_ref, st_ref, cnt_ref = refs
            agg = agg_ref[0:N, 0:cout] + agg_ref[NP:NP + N, 0:cout]
            cnt = (agg_ref[0:N, WD - 1:WD]
                   + agg_ref[NP:NP + N, WD - 1:WD]) + 1.0
            cnt_ref[...] = cnt
        else:
            agg_ref, ms_ref, cnt_in_ref, b_ref, y_ref, st_ref = refs
            agg = agg_ref[0:N, 0:cout] + agg_ref[NP:NP + N, 0:cout]
            cnt = cnt_in_ref[...]
        y = (agg + ms_ref[...]) / cnt + b_ref[...]
        mu = jnp.mean(y, axis=0, keepdims=True)
        var = jnp.mean((y - mu) ** 2, axis=0, keepdims=True)
        y_ref[...] = y
        st_ref[...] = jnp.concatenate([mu, lax.rsqrt(var + EPSV)], axis=0)

    outs = [
        jax.ShapeDtypeStruct((N, cout), jnp.float32),
        jax.ShapeDtypeStruct((2, cout), jnp.float32),
    ]
    if with_cnt:
        outs.append(jax.ShapeDtypeStruct((N, 1), jnp.float32))
    return pl.pallas_call(body, out_shape=tuple(outs))


def _f_body(y_ref, st_ref, g_ref, be_ref, W1_ref, b1_ref, W2_ref, b2_ref,
            out_ref):
    y = y_ref[...]
    h = jnp.maximum(
        (y - st_ref[0:1, :]) * st_ref[1:2, :] * g_ref[...] + be_ref[...], 0.0)
    h = jnp.maximum(_dot(h, W1_ref[...]) + b1_ref[...], 0.0)
    out_ref[...] = _dot(h, W2_ref[...]) + b2_ref[...]


def _f_call(y, st, g, be, W1, b1, W2, b2):
    return pl.pallas_call(
        _f_body,
        out_shape=jax.ShapeDtypeStruct((N, 2), jnp.float32),
    )(y, st, g, be, W1, b1, W2, b2)


@functools.lru_cache(maxsize=None)
def _sc_call():
    mesh = plsc.VectorSubcoreMesh(core_axis_name="c", subcore_axis_name="s")
    ngroups = EPT // 16

    @functools.partial(
        pl.kernel,
        mesh=mesh,
        out_type=jax.ShapeDtypeStruct((2 * NP, WD), jnp.float32),
        scratch_types=[
            pltpu.VMEM((EPT,), jnp.int32),
            pltpu.VMEM((EPT,), jnp.int32),
            pltpu.VMEM((EPT,), jnp.float32),
            pltpu.VMEM((16, CSC), jnp.float32),
            pltpu.VMEM((16, CSC), jnp.float32),
            pltpu.VMEM((16, MW), jnp.float32),
            pltpu.VMEM((16, WD), jnp.float32),
            pltpu.VMEM((8, 128), jnp.float32),
            pltpu.VMEM((16,), jnp.int32),
            pltpu.VMEM((16,), jnp.int32),
            pltpu.VMEM((16,), jnp.int32),
            pltpu.VMEM_SHARED((NP, WD), jnp.float32),
            pltpu.SemaphoreType.DMA,
            pltpu.SemaphoreType.DMA,
            pltpu.SemaphoreType.DMA,
        ],
    )
    def sck(m_hbm, un_hbm, cb_hbm, src_hbm, dst_hbm, wgt_hbm, out_hbm,
            src_v, dst_v, wgt_v, ua_v, ub_v, mrow_v, msg_v, c_v,
            gi_v, si_v, di_v, agg_sh, sem_a, sem_b, sem_m):
        cid = lax.axis_index("c")
        sid = lax.axis_index("s")
        ebase = (cid * NSUB + sid) * EPT
        pltpu.sync_copy(src_hbm.at[pl.ds(ebase, EPT)], src_v)
        pltpu.sync_copy(dst_hbm.at[pl.ds(ebase, EPT)], dst_v)
        pltpu.sync_copy(wgt_hbm.at[pl.ds(ebase, EPT)], wgt_v)
        pltpu.sync_copy(cb_hbm, c_v)
        zero = jnp.zeros((16,), jnp.float32)
        for r in range(16):
            for j in range(WD // 16):
                msg_v[r, pl.ds(j * 16, 16)] = zero
        row0 = sid * ROWS_PER_TILE
        for k in range(ROWS_PER_TILE // 16):
            pltpu.sync_copy(msg_v, agg_sh.at[pl.ds(row0 + k * 16, 16)])
        zrem = ROWS_PER_TILE % 16
        if zrem:
            pltpu.sync_copy(
                msg_v.at[pl.ds(0, zrem)],
                agg_sh.at[pl.ds(row0 + ROWS_PER_TILE - zrem, zrem)])
        plsc.subcore_barrier()

        lanes = jnp.arange(16, dtype=jnp.int32)
        cvec = c_v[0, pl.ds(0, 16)]

        def group(g, carry):
            base = g * 16
            src16 = src_v[pl.ds(base, 16)]
            dst16 = dst_v[pl.ds(base, 16)]
            w16 = wgt_v[pl.ds(base, 16)]
            gi_v[...] = src16
            si_v[...] = dst16
            di_v[...] = jnp.minimum(dst16, N - 1)
            cpa = pltpu.async_copy(un_hbm.at[gi_v], ua_v, sem_a)
            cpb = pltpu.async_copy(un_hbm.at[di_v], ub_v, sem_b)
            cpm = pltpu.async_copy(m_hbm.at[gi_v], mrow_v, sem_m)
            cpa.wait()
            cpb.wait()
            cpm.wait()

            def edge(e, ecarry):
                diff = ua_v[e, pl.ds(0, 16)] - ub_v[e, pl.ds(0, 16)] + cvec
                v = jnp.where(lanes < 8, diff, -1e30)
                mx = v
                for b in (1, 2, 4):
                    mx = jnp.maximum(mx, jnp.take_along_axis(mx, lanes ^ b,
                                                             axis=0))
                t = jnp.maximum((v - mx) * 1.4426950408889634, -126.0)
                ki = t.astype(jnp.int32)
                u = (t - ki.astype(jnp.float32)) * 0.6931471805599453
                p = 1.0 + u * (1.0 + u * (0.5 + u * (
                    0.16666666666666666 + u * (0.041666666666666664 + u * (
                        0.008333333333333333 + u * (
                            0.001388888888888889 + u * (
                                0.0001984126984126984
                                + u * 2.48015873015873e-05)))))))
                m = -ki
                scale = jnp.where((m & 1) != 0, 0.5, 1.0)
                for bit, c in ((2, 2.0 ** -2), (4, 2.0 ** -4), (8, 2.0 ** -8),
                               (16, 2.0 ** -16), (32, 2.0 ** -32),
                               (64, 2.0 ** -64)):
                    scale = scale * jnp.where((m & bit) != 0, c, 1.0)
                ex = scale * p
                ssum = ex
                for b in (1, 2, 4):
                    ssum = ssum + jnp.take_along_axis(ssum, lanes ^ b, axis=0)
                wb = jnp.take_along_axis(w16, jnp.full((16,), e, jnp.int32),
                                         axis=0)
                q_row = ex * (wb / ssum)
                qb = [jnp.take_along_axis(q_row, jnp.full((16,), h, jnp.int32),
                                          axis=0)
                      for h in range(H)]
                for j in range(CSC // 16):
                    acc = qb[0] * mrow_v[e, pl.ds(j * 16, 16)]
                    for h in range(1, H):
                        acc = acc + qb[h] * mrow_v[e, pl.ds(h * CSC + j * 16, 16)]
                    msg_v[e, pl.ds(j * 16, 16)] = acc
                return ecarry

            lax.fori_loop(0, 16, edge, 0)
            pltpu.sync_copy(msg_v, agg_sh.at[si_v], add=True)
            return carry

        lax.fori_loop(0, ngroups, group, 0)
        plsc.subcore_barrier()
        out0 = cid * NP + row0
        step = 128
        for k in range(ROWS_PER_TILE // step):
            pltpu.sync_copy(agg_sh.at[pl.ds(row0 + k * step, step)],
                            out_hbm.at[pl.ds(out0 + k * step, step)])
        drem = ROWS_PER_TILE % step
        if drem:
            done = ROWS_PER_TILE - drem
            pltpu.sync_copy(agg_sh.at[pl.ds(row0 + done, drem)],
                            out_hbm.at[pl.ds(out0 + done, drem)])

    return sck


def _conv_weights(params, i):
    p = params[f'conv{i}']
    cin, cout = CDIMS[i]
    W = p['W']
    if cout == 256:
        W = W.reshape(cin, H, 2, CSC).transpose(0, 2, 1, 3).reshape(cin, 2 * MW)
    else:
        W = W.reshape(cin, H, cout)
        W = jnp.concatenate(
            [W, jnp.zeros((cin, H, CSC - cout), jnp.float32)], axis=2)
        W = W.reshape(cin, MW)
    Up = jnp.concatenate([p['U'], jnp.zeros((cin, 8), jnp.float32)], axis=1)
    cp = jnp.concatenate([p['c'][None, :], jnp.zeros((1, 120), jnp.float32)],
                         axis=1)
    cb = jnp.broadcast_to(cp, (8, 128))
    return W, Up, cb, p['c'][None, :]


def _layout_edges(src, dst):
    i = jnp.arange(E, dtype=jnp.int32)
    order0 = jnp.argsort(dst)
    d_s = dst[order0]
    s_s = src[order0]
    first = jnp.searchsorted(d_s, d_s, side='left').astype(jnp.int32)
    r = i - first
    key = r * N + d_s
    order1 = jnp.argsort(key)
    s2 = s_s[order1]
    d2 = d_s[order1]
    r2 = r[order1]
    cnt_r = jnp.bincount(r, length=RMAX).astype(jnp.int32)
    ends = jnp.cumsum(cnt_r)
    starts = (ends - cnt_r).astype(jnp.int32)
    padded = 16 * ((cnt_r + 15) // 16)
    bases = jnp.concatenate(
        [jnp.zeros((1,), jnp.int32),
         jnp.cumsum(padded)[:-1].astype(jnp.int32)])
    pos = bases[r2] + (i - starts[r2])
    fill_dst = (N + jnp.arange(EPAD, dtype=jnp.int32) % 96)
    src_p = jnp.zeros((EPAD,), jnp.int32).at[pos].set(s2)
    dst_p = fill_dst.at[pos].set(d2)
    wgt_p = jnp.zeros((EPAD,), jnp.float32).at[pos].set(1.0)
    return src_p, dst_p, wgt_p


def kernel(pos, x, edge_index, params):
    src = edge_index[0].astype(jnp.int32)
    dst = edge_index[1].astype(jnp.int32)
    src_p, dst_p, wgt_p = _layout_edges(src, dst)

    Wc, Up, cb, c8 = _conv_weights(params, 0)
    M, un, ms = _pre_call(
        pos, x, params['norm0']['gamma'][None, :],
        params['norm0']['beta'][None, :], params['lin0_W'],
        params['lin0_b'][None, :], Wc, Up, c8)
    Ms = (M,)

    sck = _sc_call()
    cnt = None
    out = None
    for i in range(6):
        cout = CDIMS[i][1]
        b = params[f'conv{i}']['b'][None, :]
        res = [sck(Mi, un, cb, src_p, dst_p, wgt_p) for Mi in Ms]
        if i == 0:
            y, st, cnt = _p_call(cout, True)(res[0], ms, b)
        elif cout == 256:
            y, st = _p_call(cout, False)(res[0], res[1], ms, cnt, b)
        else:
            y, st = _p_call(cout, False)(res[0], ms, cnt, b)
        nrm = params[f'normc{i}']
        g = nrm['gamma'][None, :]
        be = nrm['beta'][None, :]
        if i < 5:
            cnext = CDIMS[i + 1][1]
            Wc, Up, cb, c8 = _conv_weights(params, i + 1)
            outs = _q_call(cout, cnext)(y, st, g, be, Wc, Up, c8)
            if cnext == 256:
                Ma, Mb, un, ms = outs
                Ms = (Ma, Mb)
            else:
                M, un, ms = outs
                Ms = (M,)
        else:
            out = _f_call(y, st, g, be, params['lin1_W'],
                          params['lin1_b'][None, :], params['lin2_W'],
                          params['lin2_b'][None, :])
    return out
